# Initial kernel scaffold; baseline (speedup 1.0000x reference)
import jax
import jax.numpy as jnp
from jax.experimental import pallas as pl


def kernel(x, edge_index, edge_attr, W1, b1, W2, b2, W3, b3):
    raise NotImplementedError("write your pallas kernel here")


# speedup vs baseline: 5.1809x; 5.1809x over previous
import functools

import jax
import jax.numpy as jnp
from jax import lax
from jax.experimental import pallas as pl
from jax.experimental.pallas import tpu as pltpu
from jax.experimental.pallas import tpu_sc as plsc

_N, _E, _D, _DE, _H = 10000, 320000, 128, 16, 32

_NC, _NS = 2, 16
_NW = _NC * _NS
_CH = 128
_EC = _E // _CH
_JMAX = (_EC + _NW - 1) // _NW
_NP = 10240
_RPT = _NP // _NS

_f32 = jnp.float32


def _pre_nodes_body(x_ref, w1at_ref, w1bt_ref, b1_ref, p_ref, q_ref):
    xb = x_ref[...]
    p_ref[...] = jnp.dot(xb, w1at_ref[...], preferred_element_type=_f32)
    q_ref[...] = (jnp.dot(xb, w1bt_ref[...], preferred_element_type=_f32)
                  + b1_ref[...])


def _pre_edges_body(ea_ref, w1ct_ref, a_ref):
    a_ref[...] = jnp.dot(ea_ref[...], w1ct_ref[...], preferred_element_type=_f32)


def _sc_edge_body(p_hbm, q_hbm, a_hbm, src_hbm, dst_hbm,
                  hp_hbm, cp_hbm,
                  srcv, dstv, pbuf, qbuf, abuf, onesv, zbuf, zcnt,
                  acc, cacc, sem1, sem2):
    cid = lax.axis_index("c")
    sid = lax.axis_index("s")
    wid = sid * _NC + cid
    zero16 = jnp.zeros((16,), _f32)
    one16 = jnp.ones((16,), _f32)

    def _zrow(r, c):
        zbuf[r, 0:16] = zero16
        zbuf[r, 16:32] = zero16
        return c
    lax.fori_loop(0, _RPT, _zrow, 0)

    def _zcnt(i, c):
        zcnt[pl.ds(i * 16, 16)] = zero16
        return c
    lax.fori_loop(0, _RPT // 16, _zcnt, 0)

    def _ones(i, c):
        onesv[pl.ds(i * 16, 16)] = one16
        return c
    lax.fori_loop(0, _CH // 16, _ones, 0)

    pltpu.sync_copy(zbuf, acc.at[pl.ds(sid * _RPT, _RPT)])
    pltpu.sync_copy(zcnt, cacc.at[pl.ds(sid * _RPT, _RPT)])
    plsc.subcore_barrier()

    def _chunk(j, c):
        cidx = j * _NW + wid

        @pl.when(cidx < _EC)
        def _():
            base = cidx * _CH
            pltpu.sync_copy(src_hbm.at[pl.ds(base, _CH)], srcv)
            pltpu.sync_copy(dst_hbm.at[pl.ds(base, _CH)], dstv)
            g1 = pltpu.async_copy(p_hbm.at[srcv], pbuf, sem1)
            g2 = pltpu.async_copy(q_hbm.at[dstv], qbuf, sem2)
            pltpu.sync_copy(a_hbm.at[pl.ds(base, _CH)], abuf)
            g1.wait()
            g2.wait()

            def _row(r, cc):
                h0 = jnp.maximum(pbuf[r, 0:16] + qbuf[r, 0:16] + abuf[r, 0:16],
                                 0.0)
                h1 = jnp.maximum(pbuf[r, 16:32] + qbuf[r, 16:32]
                                 + abuf[r, 16:32], 0.0)
                abuf[r, 0:16] = h0
                abuf[r, 16:32] = h1
                return cc
            lax.fori_loop(0, _CH, _row, 0)

            pltpu.sync_copy(abuf, acc.at[dstv], add=True)
            pltpu.sync_copy(onesv, cacc.at[dstv], add=True)
        return c

    lax.fori_loop(0, _JMAX, _chunk, 0)
    plsc.subcore_barrier()

    pltpu.sync_copy(acc.at[pl.ds(sid * _RPT, _RPT)],
                    hp_hbm.at[cid, pl.ds(sid * _RPT, _RPT)])
    pltpu.sync_copy(cacc.at[pl.ds(sid * _RPT, _RPT)],
                    cp_hbm.at[cid, pl.ds(sid * _RPT, _RPT)])


def _post_body(x_ref, hp_ref, cp_ref, w2t_ref, b2_ref, w3at_ref, w3bt_ref,
               b3_ref, o_ref):
    hagg = hp_ref[0] + hp_ref[1]
    cnt = cp_ref[0] + cp_ref[1]
    agg = ((jnp.dot(hagg, w2t_ref[...], preferred_element_type=_f32)
            + cnt * b2_ref[...]) / (cnt + 1e-6))
    xb = x_ref[...]
    up = (jnp.dot(xb, w3at_ref[...], preferred_element_type=_f32)
          + jnp.dot(agg, w3bt_ref[...], preferred_element_type=_f32)
          + b3_ref[...])
    o_ref[...] = xb + jnp.maximum(up, 0.0)


_BN = 1000
_BE = 8000


def kernel(x, edge_index, edge_attr, W1, b1, W2, b2, W3, b3):
    w1at = W1[:, :_D].T
    w1bt = W1[:, _D:2 * _D].T
    w1ct = W1[:, 2 * _D:].T
    w2t = W2.T
    w3at = W3[:, :_D].T
    w3bt = W3[:, _D:].T
    b1r = b1.reshape(1, _H)
    b2r = b2.reshape(1, _H)
    b3r = b3.reshape(1, _D)
    src = edge_index[0]
    dst = edge_index[1]

    p, q = pl.pallas_call(
        _pre_nodes_body,
        grid=(_N // _BN,),
        in_specs=[
            pl.BlockSpec((_BN, _D), lambda i: (i, 0)),
            pl.BlockSpec((_D, _H), lambda i: (0, 0)),
            pl.BlockSpec((_D, _H), lambda i: (0, 0)),
            pl.BlockSpec((1, _H), lambda i: (0, 0)),
        ],
        out_specs=[pl.BlockSpec((_BN, _H), lambda i: (i, 0)),
                   pl.BlockSpec((_BN, _H), lambda i: (i, 0))],
        out_shape=[jax.ShapeDtypeStruct((_N, _H), _f32),
                   jax.ShapeDtypeStruct((_N, _H), _f32)],
    )(x, w1at, w1bt, b1r)

    a = pl.pallas_call(
        _pre_edges_body,
        grid=(_E // _BE,),
        in_specs=[
            pl.BlockSpec((_BE, _DE), lambda i: (i, 0)),
            pl.BlockSpec((_DE, _H), lambda i: (0, 0)),
        ],
        out_specs=pl.BlockSpec((_BE, _H), lambda i: (i, 0)),
        out_shape=jax.ShapeDtypeStruct((_E, _H), _f32),
    )(edge_attr, w1ct)

    sc_edge = functools.partial(
        pl.kernel,
        out_type=[jax.ShapeDtypeStruct((_NC, _NP, _H), _f32),
                  jax.ShapeDtypeStruct((_NC, _NP), _f32)],
        mesh=plsc.VectorSubcoreMesh(core_axis_name="c", subcore_axis_name="s"),
        scratch_types=[
            pltpu.VMEM((_CH,), jnp.int32),
            pltpu.VMEM((_CH,), jnp.int32),
            pltpu.VMEM((_CH, _H), _f32),
            pltpu.VMEM((_CH, _H), _f32),
            pltpu.VMEM((_CH, _H), _f32),
            pltpu.VMEM((_CH,), _f32),
            pltpu.VMEM((_RPT, _H), _f32),
            pltpu.VMEM((_RPT,), _f32),
            pltpu.VMEM_SHARED((_NP, _H), _f32),
            pltpu.VMEM_SHARED((_NP,), _f32),
            pltpu.SemaphoreType.DMA,
            pltpu.SemaphoreType.DMA,
        ],
        compiler_params=pltpu.CompilerParams(use_tc_tiling_on_sc=False),
    )(_sc_edge_body)

    hp, cp = sc_edge(p, q, a, src, dst)

    out = pl.pallas_call(
        _post_body,
        grid=(_N // _BN,),
        in_specs=[
            pl.BlockSpec((_BN, _D), lambda i: (i, 0)),
            pl.BlockSpec((_NC, _BN, _H), lambda i: (0, i, 0)),
            pl.BlockSpec((_NC, _BN, 1), lambda i: (0, i, 0)),
            pl.BlockSpec((_H, _H), lambda i: (0, 0)),
            pl.BlockSpec((1, _H), lambda i: (0, 0)),
            pl.BlockSpec((_D, _D), lambda i: (0, 0)),
            pl.BlockSpec((_H, _D), lambda i: (0, 0)),
            pl.BlockSpec((1, _D), lambda i: (0, 0)),
        ],
        out_specs=pl.BlockSpec((_BN, _D), lambda i: (i, 0)),
        out_shape=jax.ShapeDtypeStruct((_N, _D), _f32),
    )(x, hp, cp.reshape(_NC, _NP, 1), w2t, b2r, w3at, w3bt, b3r)

    return out


# speedup vs baseline: 5.2872x; 1.0205x over previous
import functools

import jax
import jax.numpy as jnp
from jax import lax
from jax.experimental import pallas as pl
from jax.experimental.pallas import tpu as pltpu
from jax.experimental.pallas import tpu_sc as plsc

_N, _E, _D, _DE, _H = 10000, 320000, 128, 16, 32

_NC, _NS = 2, 16
_NW = _NC * _NS
_CH = 128
_EC = _E // _CH
_JMAX = (_EC + _NW - 1) // _NW
_NP = 10240
_RPT = _NP // _NS

_f32 = jnp.float32


def _pre_nodes_body(x_ref, w1at_ref, w1bt_ref, b1_ref, p_ref, q_ref):
    xb = x_ref[...]
    p_ref[...] = jnp.dot(xb, w1at_ref[...], preferred_element_type=_f32)
    q_ref[...] = (jnp.dot(xb, w1bt_ref[...], preferred_element_type=_f32)
                  + b1_ref[...])


_BEB = 2560
_B8 = _BEB // 8


def _pre_edges_body(eat_ref, w1ct_ref, a0_ref, a1_ref):
    t = jnp.dot(eat_ref[...].T, w1ct_ref[...], preferred_element_type=_f32)
    a0_ref[...] = jnp.concatenate(
        [t[0:_B8], t[_B8:2 * _B8], t[2 * _B8:3 * _B8], t[3 * _B8:4 * _B8]],
        axis=1)
    a1_ref[...] = jnp.concatenate(
        [t[4 * _B8:5 * _B8], t[5 * _B8:6 * _B8], t[6 * _B8:7 * _B8],
         t[7 * _B8:8 * _B8]], axis=1)


def _sc_edge_body(p_hbm, q_hbm, a0_hbm, a1_hbm, src_hbm, dst_hbm,
                  hp_hbm, cp_hbm,
                  srcv, dstv, pbuf, qbuf, abuf, hbuf, onesv, zbuf, zcnt,
                  acc, cacc, sem1, sem2):
    cid = lax.axis_index("c")
    sid = lax.axis_index("s")
    wid = sid * _NC + cid
    zero16 = jnp.zeros((16,), _f32)
    one16 = jnp.ones((16,), _f32)

    def _zrow(r, c):
        zbuf[r, 0:16] = zero16
        zbuf[r, 16:32] = zero16
        return c
    lax.fori_loop(0, _RPT, _zrow, 0)

    def _zcnt(i, c):
        zcnt[pl.ds(i * 16, 16)] = zero16
        return c
    lax.fori_loop(0, _RPT // 16, _zcnt, 0)

    def _ones(i, c):
        onesv[pl.ds(i * 16, 16)] = one16
        return c
    lax.fori_loop(0, _CH // 16, _ones, 0)

    pltpu.sync_copy(zbuf, acc.at[pl.ds(sid * _RPT, _RPT)])
    pltpu.sync_copy(zcnt, cacc.at[pl.ds(sid * _RPT, _RPT)])
    plsc.subcore_barrier()

    def _chunk(j, c):
        cidx = j * _NW + wid

        @pl.when(cidx < _EC)
        def _():
            base = cidx * _CH
            pltpu.sync_copy(src_hbm.at[pl.ds(base, _CH)], srcv)
            pltpu.sync_copy(dst_hbm.at[pl.ds(base, _CH)], dstv)
            g1 = pltpu.async_copy(p_hbm.at[srcv], pbuf, sem1)
            g2 = pltpu.async_copy(q_hbm.at[dstv], qbuf, sem2)
            pltpu.sync_copy(a0_hbm.at[pl.ds(cidx * 16, 16)],
                            abuf.at[pl.ds(0, 16)])
            pltpu.sync_copy(a1_hbm.at[pl.ds(cidx * 16, 16)],
                            abuf.at[pl.ds(16, 16)])
            g1.wait()
            g2.wait()

            def _row(r, cc):
                for k in range(4):
                    e = 4 * r + k
                    for hh in range(2):
                        col = 32 * k + 16 * hh
                        v = (pbuf[e, pl.ds(16 * hh, 16)]
                             + qbuf[e, pl.ds(16 * hh, 16)]
                             + abuf[r, pl.ds(col, 16)])
                        hbuf[e, pl.ds(16 * hh, 16)] = jnp.maximum(v, 0.0)
                return cc
            lax.fori_loop(0, 32, _row, 0)

            pltpu.sync_copy(hbuf, acc.at[dstv], add=True)
            pltpu.sync_copy(onesv, cacc.at[dstv], add=True)
        return c

    lax.fori_loop(0, _JMAX, _chunk, 0)
    plsc.subcore_barrier()

    pltpu.sync_copy(acc.at[pl.ds(sid * _RPT, _RPT)],
                    hp_hbm.at[cid, pl.ds(sid * _RPT, _RPT)])
    pltpu.sync_copy(cacc.at[pl.ds(sid * _RPT, _RPT)],
                    cp_hbm.at[cid, pl.ds(sid * _RPT, _RPT)])


def _post_body(x_ref, hp_ref, cp_ref, w2t_ref, b2_ref, w3at_ref, w3bt_ref,
               b3_ref, o_ref):
    hagg = hp_ref[0] + hp_ref[1]
    cnt = cp_ref[0] + cp_ref[1]
    agg = ((jnp.dot(hagg, w2t_ref[...], preferred_element_type=_f32)
            + cnt * b2_ref[...]) / (cnt + 1e-6))
    xb = x_ref[...]
    up = (jnp.dot(xb, w3at_ref[...], preferred_element_type=_f32)
          + jnp.dot(agg, w3bt_ref[...], preferred_element_type=_f32)
          + b3_ref[...])
    o_ref[...] = xb + jnp.maximum(up, 0.0)


_BN = 1000
_BE = 8000


def kernel(x, edge_index, edge_attr, W1, b1, W2, b2, W3, b3):
    w1at = W1[:, :_D].T
    w1bt = W1[:, _D:2 * _D].T
    w1ct = W1[:, 2 * _D:].T
    w2t = W2.T
    w3at = W3[:, :_D].T
    w3bt = W3[:, _D:].T
    b1r = b1.reshape(1, _H)
    b2r = b2.reshape(1, _H)
    b3r = b3.reshape(1, _D)
    def _perm(v):
        return (v.reshape(-1, 2, 4, _B8).transpose(0, 3, 1, 2)
                .reshape(-1, 16, 2, 4).swapaxes(1, 2).reshape(_E))
    src = _perm(edge_index[0])
    dst = _perm(edge_index[1])
    eat = edge_attr.T

    p, q = pl.pallas_call(
        _pre_nodes_body,
        grid=(_N // _BN,),
        in_specs=[
            pl.BlockSpec((_BN, _D), lambda i: (i, 0)),
            pl.BlockSpec((_D, _H), lambda i: (0, 0)),
            pl.BlockSpec((_D, _H), lambda i: (0, 0)),
            pl.BlockSpec((1, _H), lambda i: (0, 0)),
        ],
        out_specs=[pl.BlockSpec((_BN, _H), lambda i: (i, 0)),
                   pl.BlockSpec((_BN, _H), lambda i: (i, 0))],
        out_shape=[jax.ShapeDtypeStruct((_N, _H), _f32),
                   jax.ShapeDtypeStruct((_N, _H), _f32)],
    )(x, w1at, w1bt, b1r)

    a0, a1 = pl.pallas_call(
        _pre_edges_body,
        grid=(_E // _BEB,),
        in_specs=[
            pl.BlockSpec((_DE, _BEB), lambda i: (0, i)),
            pl.BlockSpec((_DE, _H), lambda i: (0, 0)),
        ],
        out_specs=[pl.BlockSpec((_B8, 128), lambda i: (i, 0)),
                   pl.BlockSpec((_B8, 128), lambda i: (i, 0))],
        out_shape=[jax.ShapeDtypeStruct((_E // 8, 128), _f32),
                   jax.ShapeDtypeStruct((_E // 8, 128), _f32)],
    )(eat, w1ct)

    sc_edge = functools.partial(
        pl.kernel,
        out_type=[jax.ShapeDtypeStruct((_NC, _NP, _H), _f32),
                  jax.ShapeDtypeStruct((_NC, _NP), _f32)],
        mesh=plsc.VectorSubcoreMesh(core_axis_name="c", subcore_axis_name="s"),
        scratch_types=[
            pltpu.VMEM((_CH,), jnp.int32),
            pltpu.VMEM((_CH,), jnp.int32),
            pltpu.VMEM((_CH, _H), _f32),
            pltpu.VMEM((_CH, _H), _f32),
            pltpu.VMEM((32, 128), _f32),
            pltpu.VMEM((_CH, _H), _f32),
            pltpu.VMEM((_CH,), _f32),
            pltpu.VMEM((_RPT, _H), _f32),
            pltpu.VMEM((_RPT,), _f32),
            pltpu.VMEM_SHARED((_NP, _H), _f32),
            pltpu.VMEM_SHARED((_NP,), _f32),
            pltpu.SemaphoreType.DMA,
            pltpu.SemaphoreType.DMA,
        ],
        compiler_params=pltpu.CompilerParams(use_tc_tiling_on_sc=False),
    )(_sc_edge_body)

    hp, cp = sc_edge(p, q, a0, a1, src, dst)

    out = pl.pallas_call(
        _post_body,
        grid=(_N // _BN,),
        in_specs=[
            pl.BlockSpec((_BN, _D), lambda i: (i, 0)),
            pl.BlockSpec((_NC, _BN, _H), lambda i: (0, i, 0)),
            pl.BlockSpec((_NC, _BN, 1), lambda i: (0, i, 0)),
            pl.BlockSpec((_H, _H), lambda i: (0, 0)),
            pl.BlockSpec((1, _H), lambda i: (0, 0)),
            pl.BlockSpec((_D, _D), lambda i: (0, 0)),
            pl.BlockSpec((_H, _D), lambda i: (0, 0)),
            pl.BlockSpec((1, _D), lambda i: (0, 0)),
        ],
        out_specs=pl.BlockSpec((_BN, _D), lambda i: (i, 0)),
        out_shape=jax.ShapeDtypeStruct((_N, _D), _f32),
    )(x, hp, cp.reshape(_NC, _NP, 1), w2t, b2r, w3at, w3bt, b3r)

    return out


# speedup vs baseline: 7.0337x; 1.3303x over previous
import functools

import jax
import jax.numpy as jnp
from jax import lax
from jax.experimental import pallas as pl
from jax.experimental.pallas import tpu as pltpu
from jax.experimental.pallas import tpu_sc as plsc

_N, _E, _D, _DE, _H = 10000, 320000, 128, 16, 32

_NC, _NS = 2, 16
_NW = _NC * _NS
_CH = 128
_EC = _E // _CH
_JMAX = (_EC + _NW - 1) // _NW
_NP = 10240
_RPT = _NP // _NS

_f32 = jnp.float32


def _pre_nodes_body(x_ref, w1at_ref, w1bt_ref, b1_ref, p_ref, q_ref):
    xb = x_ref[...]
    p_ref[...] = jnp.dot(xb, w1at_ref[...], preferred_element_type=_f32)
    q_ref[...] = (jnp.dot(xb, w1bt_ref[...], preferred_element_type=_f32)
                  + b1_ref[...])


_BEB = 2560
_B8 = _BEB // 8


def _pre_edges_body(eat_ref, w1ct_ref, a0_ref, a1_ref):
    t = lax.dot_general(eat_ref[...], w1ct_ref[...],
                        (((0,), (0,)), ((), ())),
                        preferred_element_type=_f32)
    a0_ref[...] = jnp.concatenate(
        [t[0:_B8], t[_B8:2 * _B8], t[2 * _B8:3 * _B8], t[3 * _B8:4 * _B8]],
        axis=1)
    a1_ref[...] = jnp.concatenate(
        [t[4 * _B8:5 * _B8], t[5 * _B8:6 * _B8], t[6 * _B8:7 * _B8],
         t[7 * _B8:8 * _B8]], axis=1)


def _sc_edge_body(p_hbm, q_hbm, a0_hbm, a1_hbm, src_hbm, dst_hbm,
                  hp_hbm, cp_hbm,
                  srcv, dstv, pbuf, qbuf, abuf, hbuf, onesv, zbuf, zcnt,
                  acc, cacc, sem1, sem2):
    cid = lax.axis_index("c")
    sid = lax.axis_index("s")
    wid = sid * _NC + cid
    zero16 = jnp.zeros((16,), _f32)
    one16 = jnp.ones((16,), _f32)

    def _zrow(r, c):
        zbuf[r, 0:16] = zero16
        zbuf[r, 16:32] = zero16
        return c
    lax.fori_loop(0, _RPT, _zrow, 0)

    def _zcnt(i, c):
        zcnt[pl.ds(i * 16, 16)] = zero16
        return c
    lax.fori_loop(0, _RPT // 16, _zcnt, 0)

    def _ones(i, c):
        onesv[pl.ds(i * 16, 16)] = one16
        return c
    lax.fori_loop(0, _CH // 16, _ones, 0)

    pltpu.sync_copy(zbuf, acc.at[pl.ds(sid * _RPT, _RPT)])
    pltpu.sync_copy(zcnt, cacc.at[pl.ds(sid * _RPT, _RPT)])
    plsc.subcore_barrier()

    def _chunk(j, c):
        cidx = j * _NW + wid

        @pl.when(cidx < _EC)
        def _():
            b = (cidx * 52429) >> 20
            ebase = 2560 * b + 16 * (cidx - 20 * b)
            idx_cps = []
            for half in range(2):
                for k in range(4):
                    off = ebase + 1280 * half + 320 * k
                    pos = 16 * (4 * half + k)
                    idx_cps.append(pltpu.async_copy(
                        src_hbm.at[pl.ds(off, 16)],
                        srcv.at[pl.ds(pos, 16)], sem1))
                    idx_cps.append(pltpu.async_copy(
                        dst_hbm.at[pl.ds(off, 16)],
                        dstv.at[pl.ds(pos, 16)], sem1))
            a_cp0 = pltpu.async_copy(a0_hbm.at[pl.ds(cidx * 16, 16)],
                                     abuf.at[pl.ds(0, 16)], sem2)
            a_cp1 = pltpu.async_copy(a1_hbm.at[pl.ds(cidx * 16, 16)],
                                     abuf.at[pl.ds(16, 16)], sem2)
            for cp in idx_cps:
                cp.wait()
            g1 = pltpu.async_copy(p_hbm.at[srcv], pbuf, sem1)
            g2 = pltpu.async_copy(q_hbm.at[dstv], qbuf, sem1)
            a_cp0.wait()
            a_cp1.wait()
            g1.wait()
            g2.wait()

            def _row(i, cc):
                for half in range(2):
                    r = 16 * half + i
                    for k in range(4):
                        e = 64 * half + 16 * k + i
                        for hh in range(2):
                            col = 32 * k + 16 * hh
                            v = (pbuf[e, pl.ds(16 * hh, 16)]
                                 + qbuf[e, pl.ds(16 * hh, 16)]
                                 + abuf[r, pl.ds(col, 16)])
                            hbuf[e, pl.ds(16 * hh, 16)] = jnp.maximum(v, 0.0)
                return cc
            lax.fori_loop(0, 16, _row, 0)

            pltpu.sync_copy(hbuf, acc.at[dstv], add=True)
            pltpu.sync_copy(onesv, cacc.at[dstv], add=True)
        return c

    lax.fori_loop(0, _JMAX, _chunk, 0)
    plsc.subcore_barrier()

    pltpu.sync_copy(acc.at[pl.ds(sid * _RPT, _RPT)],
                    hp_hbm.at[cid, pl.ds(sid * _RPT, _RPT)])
    pltpu.sync_copy(cacc.at[pl.ds(sid * _RPT, _RPT)],
                    cp_hbm.at[cid, pl.ds(sid * _RPT, _RPT)])


def _post_body(x_ref, hp_ref, cp_ref, w2t_ref, b2_ref, w3at_ref, w3bt_ref,
               b3_ref, o_ref):
    hagg = hp_ref[0] + hp_ref[1]
    cnt = cp_ref[0] + cp_ref[1]
    agg = ((jnp.dot(hagg, w2t_ref[...], preferred_element_type=_f32)
            + cnt * b2_ref[...]) / (cnt + 1e-6))
    xb = x_ref[...]
    up = (jnp.dot(xb, w3at_ref[...], preferred_element_type=_f32)
          + jnp.dot(agg, w3bt_ref[...], preferred_element_type=_f32)
          + b3_ref[...])
    o_ref[...] = xb + jnp.maximum(up, 0.0)


_BN = 1000
_BE = 8000


def kernel(x, edge_index, edge_attr, W1, b1, W2, b2, W3, b3):
    w1at = W1[:, :_D].T
    w1bt = W1[:, _D:2 * _D].T
    w1ct = W1[:, 2 * _D:].T
    w2t = W2.T
    w3at = W3[:, :_D].T
    w3bt = W3[:, _D:].T
    b1r = b1.reshape(1, _H)
    b2r = b2.reshape(1, _H)
    b3r = b3.reshape(1, _D)
    src = edge_index[0]
    dst = edge_index[1]
    eat = edge_attr.T

    p, q = pl.pallas_call(
        _pre_nodes_body,
        grid=(_N // _BN,),
        in_specs=[
            pl.BlockSpec((_BN, _D), lambda i: (i, 0)),
            pl.BlockSpec((_D, _H), lambda i: (0, 0)),
            pl.BlockSpec((_D, _H), lambda i: (0, 0)),
            pl.BlockSpec((1, _H), lambda i: (0, 0)),
        ],
        out_specs=[pl.BlockSpec((_BN, _H), lambda i: (i, 0)),
                   pl.BlockSpec((_BN, _H), lambda i: (i, 0))],
        out_shape=[jax.ShapeDtypeStruct((_N, _H), _f32),
                   jax.ShapeDtypeStruct((_N, _H), _f32)],
    )(x, w1at, w1bt, b1r)

    a0, a1 = pl.pallas_call(
        _pre_edges_body,
        grid=(_E // _BEB,),
        in_specs=[
            pl.BlockSpec((_DE, _BEB), lambda i: (0, i)),
            pl.BlockSpec((_DE, _H), lambda i: (0, 0)),
        ],
        out_specs=[pl.BlockSpec((_B8, 128), lambda i: (i, 0)),
                   pl.BlockSpec((_B8, 128), lambda i: (i, 0))],
        out_shape=[jax.ShapeDtypeStruct((_E // 8, 128), _f32),
                   jax.ShapeDtypeStruct((_E // 8, 128), _f32)],
    )(eat, w1ct)

    sc_edge = functools.partial(
        pl.kernel,
        out_type=[jax.ShapeDtypeStruct((_NC, _NP, _H), _f32),
                  jax.ShapeDtypeStruct((_NC, _NP), _f32)],
        mesh=plsc.VectorSubcoreMesh(core_axis_name="c", subcore_axis_name="s"),
        scratch_types=[
            pltpu.VMEM((_CH,), jnp.int32),
            pltpu.VMEM((_CH,), jnp.int32),
            pltpu.VMEM((_CH, _H), _f32),
            pltpu.VMEM((_CH, _H), _f32),
            pltpu.VMEM((32, 128), _f32),
            pltpu.VMEM((_CH, _H), _f32),
            pltpu.VMEM((_CH,), _f32),
            pltpu.VMEM((_RPT, _H), _f32),
            pltpu.VMEM((_RPT,), _f32),
            pltpu.VMEM_SHARED((_NP, _H), _f32),
            pltpu.VMEM_SHARED((_NP,), _f32),
            pltpu.SemaphoreType.DMA,
            pltpu.SemaphoreType.DMA,
        ],
        compiler_params=pltpu.CompilerParams(use_tc_tiling_on_sc=False),
    )(_sc_edge_body)

    hp, cp = sc_edge(p, q, a0, a1, src, dst)

    out = pl.pallas_call(
        _post_body,
        grid=(_N // _BN,),
        in_specs=[
            pl.BlockSpec((_BN, _D), lambda i: (i, 0)),
            pl.BlockSpec((_NC, _BN, _H), lambda i: (0, i, 0)),
            pl.BlockSpec((_NC, _BN, 1), lambda i: (0, i, 0)),
            pl.BlockSpec((_H, _H), lambda i: (0, 0)),
            pl.BlockSpec((1, _H), lambda i: (0, 0)),
            pl.BlockSpec((_D, _D), lambda i: (0, 0)),
            pl.BlockSpec((_H, _D), lambda i: (0, 0)),
            pl.BlockSpec((1, _D), lambda i: (0, 0)),
        ],
        out_specs=pl.BlockSpec((_BN, _D), lambda i: (i, 0)),
        out_shape=jax.ShapeDtypeStruct((_N, _D), _f32),
    )(x, hp, cp.reshape(_NC, _NP, 1), w2t, b2r, w3at, w3bt, b3r)

    return out


# speedup vs baseline: 11.3253x; 1.6102x over previous
import functools

import jax
import jax.numpy as jnp
from jax import lax
from jax.experimental import pallas as pl
from jax.experimental.pallas import tpu as pltpu
from jax.experimental.pallas import tpu_sc as plsc

_N, _E, _D, _DE, _H = 10000, 320000, 128, 16, 32

_NC, _NS = 2, 16
_NW = _NC * _NS
_CH = 128
_EC = _E // _CH
_JMAX = (_EC + _NW - 1) // _NW
_NP = 10240
_RPT = _NP // _NS

_f32 = jnp.float32


def _pre_nodes_body(x_ref, w1at_ref, w1bt_ref, b1_ref, p_ref, q_ref):
    xb = x_ref[...]
    p_ref[...] = jnp.dot(xb, w1at_ref[...], preferred_element_type=_f32)
    q_ref[...] = (jnp.dot(xb, w1bt_ref[...], preferred_element_type=_f32)
                  + b1_ref[...])


_BEB = 6400
_B8 = _BEB // 8
_CPB = _BEB // _CH


def _pre_edges_body(eat_ref, w1ct_ref, ek_ref, a0_ref, a1_ref):
    t = lax.dot_general(eat_ref[...], w1ct_ref[...],
                        (((0,), (0,)), ((), ())),
                        preferred_element_type=_f32)
    a0_ref[...] = sum(
        jnp.dot(t[k * _B8:(k + 1) * _B8], ek_ref[k],
                preferred_element_type=_f32) for k in range(4))
    a1_ref[...] = sum(
        jnp.dot(t[(4 + k) * _B8:(5 + k) * _B8], ek_ref[k],
                preferred_element_type=_f32) for k in range(4))


def _sc_edge_body(p_hbm, q_hbm, a0_hbm, a1_hbm, src_hbm, dst_hbm,
                  hp_hbm, cp_hbm,
                  srcv0, dstv0, srcv1, dstv1, srcv2, dstv2, srcv3, dstv3,
                  pbuf0, qbuf0, pbuf1, qbuf1,
                  abuf0, abuf1, hbuf0, hbuf1,
                  onesv, zbuf, zcnt, acc, cacc,
                  semidx0, semidx1, sema0, sema1,
                  semg0, semg1, semsc0, semsc1):
    cid = lax.axis_index("c")
    sid = lax.axis_index("s")
    wid = sid * _NC + cid
    zero16 = jnp.zeros((16,), _f32)
    one16 = jnp.ones((16,), _f32)

    ibufs = ((srcv0, dstv0), (srcv1, dstv1), (srcv2, dstv2), (srcv3, dstv3))
    cbufs = ((pbuf0, qbuf0, abuf0, hbuf0, semidx0, sema0, semg0, semsc0),
             (pbuf1, qbuf1, abuf1, hbuf1, semidx1, sema1, semg1, semsc1))

    def _zrow(r, c):
        zbuf[r, 0:16] = zero16
        zbuf[r, 16:32] = zero16
        return c
    lax.fori_loop(0, _RPT, _zrow, 0)

    def _zcnt(i, c):
        zcnt[pl.ds(i * 16, 16)] = zero16
        return c
    lax.fori_loop(0, _RPT // 16, _zcnt, 0)

    def _ones(i, c):
        onesv[pl.ds(i * 16, 16)] = one16
        return c
    lax.fori_loop(0, _CH // 16, _ones, 0)

    pltpu.sync_copy(zbuf, acc.at[pl.ds(sid * _RPT, _RPT)])
    pltpu.sync_copy(zcnt, cacc.at[pl.ds(sid * _RPT, _RPT)])
    plsc.subcore_barrier()

    def issue_idx(cidx, I, sem):
        srcv, dstv = I
        b = (cidx * 10486) >> 19
        ebase = _BEB * b + 16 * (cidx - _CPB * b)
        for half in range(2):
            for k in range(4):
                off = ebase + 4 * _B8 * half + _B8 * k
                pos = 16 * (4 * half + k)
                pltpu.async_copy(src_hbm.at[pl.ds(off, 16)],
                                 srcv.at[pl.ds(pos, 16)], sem)
                pltpu.async_copy(dst_hbm.at[pl.ds(off, 16)],
                                 dstv.at[pl.ds(pos, 16)], sem)

    def drain_idx(I, sem):
        pltpu.make_async_copy(src_hbm.at[pl.ds(0, _CH)], I[0], sem).wait()
        pltpu.make_async_copy(dst_hbm.at[pl.ds(0, _CH)], I[1], sem).wait()

    def issue_a(cidx, C):
        abuf = C[2]
        pltpu.async_copy(a0_hbm.at[pl.ds(cidx * 16, 16)],
                         abuf.at[pl.ds(0, 16)], C[5])
        pltpu.async_copy(a1_hbm.at[pl.ds(cidx * 16, 16)],
                         abuf.at[pl.ds(16, 16)], C[5])

    def drain_a(C):
        abuf = C[2]
        pltpu.make_async_copy(a0_hbm.at[pl.ds(0, 16)],
                              abuf.at[pl.ds(0, 16)], C[5]).wait()
        pltpu.make_async_copy(a1_hbm.at[pl.ds(0, 16)],
                              abuf.at[pl.ds(16, 16)], C[5]).wait()

    def issue_gather(I, C):
        pltpu.async_copy(p_hbm.at[I[0]], C[0], C[6])
        pltpu.async_copy(q_hbm.at[I[1]], C[1], C[6])

    def drain_gather(C):
        pltpu.make_async_copy(p_hbm.at[pl.ds(0, _CH)], C[0], C[6]).wait()
        pltpu.make_async_copy(q_hbm.at[pl.ds(0, _CH)], C[1], C[6]).wait()

    def issue_scatter(I, C):
        pltpu.async_copy(C[3], acc.at[I[1]], C[7], add=True)
        pltpu.async_copy(onesv, cacc.at[I[1]], C[7], add=True)

    def drain_scatter(I, C):
        pltpu.make_async_copy(C[3], acc.at[I[1]], C[7]).wait()
        pltpu.make_async_copy(onesv, cacc.at[I[1]], C[7]).wait()

    def valu(C):
        pbuf, qbuf, abuf, hbuf = C[0], C[1], C[2], C[3]

        def _row(i, cc):
            for half in range(2):
                r = 16 * half + i
                for k in range(4):
                    e = 64 * half + 16 * k + i
                    for hh in range(2):
                        col = 32 * k + 16 * hh
                        v = (pbuf[e, pl.ds(16 * hh, 16)]
                             + qbuf[e, pl.ds(16 * hh, 16)]
                             + abuf[r, pl.ds(col, 16)])
                        hbuf[e, pl.ds(16 * hh, 16)] = jnp.maximum(v, 0.0)
            return cc
        lax.fori_loop(0, 16, _row, 0)

    issue_idx(wid, ibufs[0], cbufs[0][4])
    issue_a(wid, cbufs[0])
    drain_idx(ibufs[0], cbufs[0][4])
    issue_gather(ibufs[0], cbufs[0])
    issue_idx(_NW + wid, ibufs[1], cbufs[1][4])
    issue_a(_NW + wid, cbufs[1])

    def _quad(jj, c):
        for par in range(4):
            par2 = par % 2
            C = cbufs[par2]
            Cn = cbufs[1 - par2]
            I = ibufs[par]
            In1 = ibufs[(par + 1) % 4]
            In2 = ibufs[(par + 2) % 4]
            Im2 = ibufs[(par + 2) % 4]
            s = 4 * jj + par
            cidx = s * _NW + wid
            act_s = cidx < _EC
            act_s1 = cidx + _NW < _EC
            act_s2 = cidx + 2 * _NW < _EC

            @pl.when(act_s)
            def _():
                drain_gather(C)
                drain_a(C)

            @pl.when(s >= 2)
            def _():
                drain_scatter(Im2, C)

            @pl.when(act_s1)
            def _():
                drain_idx(In1, Cn[4])
                issue_gather(In1, Cn)

            @pl.when(act_s2)
            def _():
                issue_idx(cidx + 2 * _NW, In2, C[4])

            @pl.when(act_s)
            def _():
                valu(C)

            @pl.when(act_s2)
            def _():
                issue_a(cidx + 2 * _NW, C)

            @pl.when(act_s)
            def _():
                issue_scatter(I, C)
        return c

    lax.fori_loop(0, 20, _quad, 0)

    @pl.when(78 * _NW + wid < _EC)
    def _():
        drain_scatter(ibufs[2], cbufs[0])

    plsc.subcore_barrier()

    pltpu.sync_copy(acc.at[pl.ds(sid * _RPT, _RPT)],
                    hp_hbm.at[cid, pl.ds(sid * _RPT, _RPT)])
    pltpu.sync_copy(cacc.at[pl.ds(sid * _RPT, _RPT)],
                    cp_hbm.at[cid, pl.ds(sid * _RPT, _RPT)])


def _post_body(x_ref, hp_ref, cp_ref, w2t_ref, b2_ref, w3at_ref, w3bt_ref,
               b3_ref, o_ref):
    hagg = hp_ref[0] + hp_ref[1]
    cnt = cp_ref[0] + cp_ref[1]
    agg = ((jnp.dot(hagg, w2t_ref[...], preferred_element_type=_f32)
            + cnt * b2_ref[...]) / (cnt + 1e-6))
    xb = x_ref[...]
    up = (jnp.dot(xb, w3at_ref[...], preferred_element_type=_f32)
          + jnp.dot(agg, w3bt_ref[...], preferred_element_type=_f32)
          + b3_ref[...])
    o_ref[...] = xb + jnp.maximum(up, 0.0)


_BN = 1000
_BE = 8000


def kernel(x, edge_index, edge_attr, W1, b1, W2, b2, W3, b3):
    w1at = W1[:, :_D].T
    w1bt = W1[:, _D:2 * _D].T
    w1ct = W1[:, 2 * _D:].T
    w2t = W2.T
    w3at = W3[:, :_D].T
    w3bt = W3[:, _D:].T
    b1r = b1.reshape(1, _H)
    b2r = b2.reshape(1, _H)
    b3r = b3.reshape(1, _D)
    src = edge_index[0]
    dst = edge_index[1]
    eat = edge_attr.T
    eye32 = jnp.eye(_H, dtype=_f32)
    ek = jnp.stack([jnp.zeros((_H, 128), _f32)
                    .at[:, 32 * k:32 * (k + 1)].set(eye32) for k in range(4)])

    p, q = pl.pallas_call(
        _pre_nodes_body,
        grid=(_N // _BN,),
        in_specs=[
            pl.BlockSpec((_BN, _D), lambda i: (i, 0)),
            pl.BlockSpec((_D, _H), lambda i: (0, 0)),
            pl.BlockSpec((_D, _H), lambda i: (0, 0)),
            pl.BlockSpec((1, _H), lambda i: (0, 0)),
        ],
        out_specs=[pl.BlockSpec((_BN, _H), lambda i: (i, 0)),
                   pl.BlockSpec((_BN, _H), lambda i: (i, 0))],
        out_shape=[jax.ShapeDtypeStruct((_N, _H), _f32),
                   jax.ShapeDtypeStruct((_N, _H), _f32)],
    )(x, w1at, w1bt, b1r)

    a0, a1 = pl.pallas_call(
        _pre_edges_body,
        grid=(_E // _BEB,),
        in_specs=[
            pl.BlockSpec((_DE, _BEB), lambda i: (0, i)),
            pl.BlockSpec((_DE, _H), lambda i: (0, 0)),
            pl.BlockSpec((4, _H, 128), lambda i: (0, 0, 0)),
        ],
        out_specs=[pl.BlockSpec((_B8, 128), lambda i: (i, 0)),
                   pl.BlockSpec((_B8, 128), lambda i: (i, 0))],
        out_shape=[jax.ShapeDtypeStruct((_E // 8, 128), _f32),
                   jax.ShapeDtypeStruct((_E // 8, 128), _f32)],
        compiler_params=pltpu.CompilerParams(
            fuse_transposed_lhs_in_matmul=True),
    )(eat, w1ct, ek)

    sc_edge = functools.partial(
        pl.kernel,
        out_type=[jax.ShapeDtypeStruct((_NC, _NP, _H), _f32),
                  jax.ShapeDtypeStruct((_NC, _NP), _f32)],
        mesh=plsc.VectorSubcoreMesh(core_axis_name="c", subcore_axis_name="s"),
        scratch_types=(
            [pltpu.VMEM((_CH,), jnp.int32)] * 8
            + [pltpu.VMEM((_CH, _H), _f32)] * 4
            + [pltpu.VMEM((32, 128), _f32)] * 2
            + [pltpu.VMEM((_CH, _H), _f32)] * 2
            + [pltpu.VMEM((_CH,), _f32),
               pltpu.VMEM((_RPT, _H), _f32),
               pltpu.VMEM((_RPT,), _f32),
               pltpu.VMEM_SHARED((_NP, _H), _f32),
               pltpu.VMEM_SHARED((_NP,), _f32)]
            + [pltpu.SemaphoreType.DMA] * 8
        ),
        compiler_params=pltpu.CompilerParams(use_tc_tiling_on_sc=False),
    )(_sc_edge_body)

    hp, cp = sc_edge(p, q, a0, a1, src, dst)

    out = pl.pallas_call(
        _post_body,
        grid=(_N // _BN,),
        in_specs=[
            pl.BlockSpec((_BN, _D), lambda i: (i, 0)),
            pl.BlockSpec((_NC, _BN, _H), lambda i: (0, i, 0)),
            pl.BlockSpec((_NC, _BN, 1), lambda i: (0, i, 0)),
            pl.BlockSpec((_H, _H), lambda i: (0, 0)),
            pl.BlockSpec((1, _H), lambda i: (0, 0)),
            pl.BlockSpec((_D, _D), lambda i: (0, 0)),
            pl.BlockSpec((_H, _D), lambda i: (0, 0)),
            pl.BlockSpec((1, _D), lambda i: (0, 0)),
        ],
        out_specs=pl.BlockSpec((_BN, _D), lambda i: (i, 0)),
        out_shape=jax.ShapeDtypeStruct((_N, _D), _f32),
    )(x, hp, cp.reshape(_NC, _NP, 1), w2t, b2r, w3at, w3bt, b3r)

    return out


# speedup vs baseline: 12.3452x; 1.0901x over previous
import functools

import jax
import jax.numpy as jnp
from jax import lax
from jax.experimental import pallas as pl
from jax.experimental.pallas import tpu as pltpu
from jax.experimental.pallas import tpu_sc as plsc

_N, _E, _D, _DE, _H = 10000, 320000, 128, 16, 32

_NC, _NS = 2, 16
_NW = _NC * _NS
_CH = 128
_EC = _E // _CH
_JMAX = (_EC + _NW - 1) // _NW
_NP = 10240
_RPT = _NP // _NS

_f32 = jnp.float32


def _pre_nodes_body(x_ref, w1at_ref, w1bt_ref, b1_ref, p_ref, q_ref):
    xb = x_ref[...]
    p_ref[...] = jnp.dot(xb, w1at_ref[...], preferred_element_type=_f32)
    q_ref[...] = (jnp.dot(xb, w1bt_ref[...], preferred_element_type=_f32)
                  + b1_ref[...])


_BEB = 12800
_B8 = _BEB // 8
_CPB = _BEB // _CH


def _pre_edges_body(eat_ref, w1ct_ref, ek_ref, a0_ref, a1_ref):
    t = lax.dot_general(eat_ref[...], w1ct_ref[...],
                        (((0,), (0,)), ((), ())),
                        preferred_element_type=_f32)
    a0_ref[...] = sum(
        jnp.dot(t[k * _B8:(k + 1) * _B8], ek_ref[k],
                preferred_element_type=_f32) for k in range(4))
    a1_ref[...] = sum(
        jnp.dot(t[(4 + k) * _B8:(5 + k) * _B8], ek_ref[k],
                preferred_element_type=_f32) for k in range(4))


def _sc_edge_body(p_hbm, q_hbm, a0_hbm, a1_hbm, ei_hbm,
                  hp_hbm, cp_hbm,
                  sivd0, sivd1, sivd2, sivd3,
                  pbuf0, qbuf0, pbuf1, qbuf1,
                  abuf0, abuf1, hbuf0, hbuf1,
                  onesv, zbuf, zcnt, acc, cacc,
                  semidx0, semidx1, sema0, sema1,
                  semg0, semg1, semsc0, semsc1):
    cid = lax.axis_index("c")
    sid = lax.axis_index("s")
    wid = sid * _NC + cid
    zero16 = jnp.zeros((16,), _f32)
    one16 = jnp.ones((16,), _f32)

    ibufs = (sivd0, sivd1, sivd2, sivd3)
    cbufs = ((pbuf0, qbuf0, abuf0, hbuf0, semidx0, sema0, semg0, semsc0),
             (pbuf1, qbuf1, abuf1, hbuf1, semidx1, sema1, semg1, semsc1))

    def _zrow(r, c):
        zbuf[r, 0:16] = zero16
        zbuf[r, 16:32] = zero16
        return c
    lax.fori_loop(0, _RPT, _zrow, 0)

    def _zcnt(i, c):
        zcnt[pl.ds(i * 16, 16)] = zero16
        return c
    lax.fori_loop(0, _RPT // 16, _zcnt, 0)

    def _ones(i, c):
        onesv[pl.ds(i * 16, 16)] = one16
        return c
    lax.fori_loop(0, _CH // 16, _ones, 0)

    pltpu.sync_copy(zbuf, acc.at[pl.ds(sid * _RPT, _RPT)])
    pltpu.sync_copy(zcnt, cacc.at[pl.ds(sid * _RPT, _RPT)])
    plsc.subcore_barrier()

    def issue_idx(cidx, I, sem):
        b = (cidx * 10486) >> 20
        ebase = _BEB * b + 16 * (cidx - _CPB * b)
        for half in range(2):
            for k in range(4):
                off = ebase + 4 * _B8 * half + _B8 * k
                pos = 16 * (4 * half + k)
                c0 = pl.multiple_of(off & 127, 16)
                pltpu.async_copy(ei_hbm.at[off >> 7, slice(None),
                                           pl.ds(c0, 16)],
                                 I.at[:, pl.ds(pos, 16)], sem)

    def drain_idx(I, sem):
        pltpu.make_async_copy(ei_hbm.at[0], I, sem).wait()

    def issue_a(cidx, C):
        abuf = C[2]
        pltpu.async_copy(a0_hbm.at[pl.ds(cidx * 16, 16)],
                         abuf.at[pl.ds(0, 16)], C[5])
        pltpu.async_copy(a1_hbm.at[pl.ds(cidx * 16, 16)],
                         abuf.at[pl.ds(16, 16)], C[5])

    def drain_a(C):
        abuf = C[2]
        pltpu.make_async_copy(a0_hbm.at[pl.ds(0, 16)],
                              abuf.at[pl.ds(0, 16)], C[5]).wait()
        pltpu.make_async_copy(a1_hbm.at[pl.ds(0, 16)],
                              abuf.at[pl.ds(16, 16)], C[5]).wait()

    def issue_gather(I, C):
        pltpu.async_copy(p_hbm.at[I.at[0]], C[0], C[6])
        pltpu.async_copy(q_hbm.at[I.at[1]], C[1], C[6])

    def drain_gather(C):
        pltpu.make_async_copy(p_hbm.at[pl.ds(0, _CH)], C[0], C[6]).wait()
        pltpu.make_async_copy(q_hbm.at[pl.ds(0, _CH)], C[1], C[6]).wait()

    def issue_scatter(I, C):
        pltpu.async_copy(C[3], acc.at[I.at[1]], C[7], add=True)
        pltpu.async_copy(onesv, cacc.at[I.at[1]], C[7], add=True)

    def drain_scatter(I, C):
        pltpu.make_async_copy(C[3], acc.at[I.at[1]], C[7]).wait()
        pltpu.make_async_copy(onesv, cacc.at[I.at[1]], C[7]).wait()

    def valu(C):
        pbuf, qbuf, abuf, hbuf = C[0], C[1], C[2], C[3]

        def _row(i, cc):
            for half in range(2):
                r = 16 * half + i
                for k in range(4):
                    e = 64 * half + 16 * k + i
                    for hh in range(2):
                        col = 32 * k + 16 * hh
                        v = (pbuf[e, pl.ds(16 * hh, 16)]
                             + qbuf[e, pl.ds(16 * hh, 16)]
                             + abuf[r, pl.ds(col, 16)])
                        hbuf[e, pl.ds(16 * hh, 16)] = jnp.maximum(v, 0.0)
            return cc
        lax.fori_loop(0, 16, _row, 0)

    issue_idx(wid, ibufs[0], cbufs[0][4])
    issue_a(wid, cbufs[0])
    drain_idx(ibufs[0], cbufs[0][4])
    issue_gather(ibufs[0], cbufs[0])
    issue_idx(_NW + wid, ibufs[1], cbufs[1][4])
    issue_a(_NW + wid, cbufs[1])

    def _quad(jj, c):
        for par in range(4):
            par2 = par % 2
            C = cbufs[par2]
            Cn = cbufs[1 - par2]
            I = ibufs[par]
            In1 = ibufs[(par + 1) % 4]
            In2 = ibufs[(par + 2) % 4]
            Im2 = ibufs[(par + 2) % 4]
            s = 4 * jj + par
            cidx = s * _NW + wid
            act_s = cidx < _EC
            act_s1 = cidx + _NW < _EC
            act_s2 = cidx + 2 * _NW < _EC

            @pl.when(act_s)
            def _():
                drain_gather(C)
                drain_a(C)

            @pl.when(s >= 2)
            def _():
                drain_scatter(Im2, C)

            @pl.when(act_s1)
            def _():
                drain_idx(In1, Cn[4])
                issue_gather(In1, Cn)

            @pl.when(act_s2)
            def _():
                issue_idx(cidx + 2 * _NW, In2, C[4])

            @pl.when(act_s)
            def _():
                valu(C)

            @pl.when(act_s2)
            def _():
                issue_a(cidx + 2 * _NW, C)

            @pl.when(act_s)
            def _():
                issue_scatter(I, C)
        return c

    lax.fori_loop(0, 20, _quad, 0)

    @pl.when(78 * _NW + wid < _EC)
    def _():
        drain_scatter(ibufs[2], cbufs[0])

    plsc.subcore_barrier()

    pltpu.sync_copy(acc.at[pl.ds(sid * _RPT, _RPT)],
                    hp_hbm.at[cid, pl.ds(sid * _RPT, _RPT)])
    pltpu.sync_copy(cacc.at[pl.ds(sid * _RPT, _RPT)],
                    cp_hbm.at[cid, pl.ds(sid * _RPT, _RPT)])


def _post_body(x_ref, hp_ref, cp_ref, w2t_ref, b2_ref, w3at_ref, w3bt_ref,
               b3_ref, o_ref):
    hagg = hp_ref[0] + hp_ref[1]
    cnt = cp_ref[0] + cp_ref[1]
    agg = ((jnp.dot(hagg, w2t_ref[...], preferred_element_type=_f32)
            + cnt * b2_ref[...]) / (cnt + 1e-6))
    xb = x_ref[...]
    up = (jnp.dot(xb, w3at_ref[...], preferred_element_type=_f32)
          + jnp.dot(agg, w3bt_ref[...], preferred_element_type=_f32)
          + b3_ref[...])
    o_ref[...] = xb + jnp.maximum(up, 0.0)


_BN = 1000
_BE = 8000


def kernel(x, edge_index, edge_attr, W1, b1, W2, b2, W3, b3):
    w1at = W1[:, :_D].T
    w1bt = W1[:, _D:2 * _D].T
    w1ct = W1[:, 2 * _D:].T
    w2t = W2.T
    w3at = W3[:, :_D].T
    w3bt = W3[:, _D:].T
    b1r = b1.reshape(1, _H)
    b2r = b2.reshape(1, _H)
    b3r = b3.reshape(1, _D)
    ei_v = edge_index.reshape(2, _E // 128, 128).transpose(1, 0, 2)
    eat = edge_attr.T
    eye32 = jnp.eye(_H, dtype=_f32)
    ek = jnp.stack([jnp.zeros((_H, 128), _f32)
                    .at[:, 32 * k:32 * (k + 1)].set(eye32) for k in range(4)])

    p, q = pl.pallas_call(
        _pre_nodes_body,
        grid=(_N // _BN,),
        in_specs=[
            pl.BlockSpec((_BN, _D), lambda i: (i, 0)),
            pl.BlockSpec((_D, _H), lambda i: (0, 0)),
            pl.BlockSpec((_D, _H), lambda i: (0, 0)),
            pl.BlockSpec((1, _H), lambda i: (0, 0)),
        ],
        out_specs=[pl.BlockSpec((_BN, _H), lambda i: (i, 0)),
                   pl.BlockSpec((_BN, _H), lambda i: (i, 0))],
        out_shape=[jax.ShapeDtypeStruct((_N, _H), _f32),
                   jax.ShapeDtypeStruct((_N, _H), _f32)],
    )(x, w1at, w1bt, b1r)

    a0, a1 = pl.pallas_call(
        _pre_edges_body,
        grid=(_E // _BEB,),
        in_specs=[
            pl.BlockSpec((_DE, _BEB), lambda i: (0, i)),
            pl.BlockSpec((_DE, _H), lambda i: (0, 0)),
            pl.BlockSpec((4, _H, 128), lambda i: (0, 0, 0)),
        ],
        out_specs=[pl.BlockSpec((_B8, 128), lambda i: (i, 0)),
                   pl.BlockSpec((_B8, 128), lambda i: (i, 0))],
        out_shape=[jax.ShapeDtypeStruct((_E // 8, 128), _f32),
                   jax.ShapeDtypeStruct((_E // 8, 128), _f32)],
        compiler_params=pltpu.CompilerParams(
            fuse_transposed_lhs_in_matmul=True),
    )(eat, w1ct, ek)

    sc_edge = functools.partial(
        pl.kernel,
        out_type=[jax.ShapeDtypeStruct((_NC, _NP, _H), _f32),
                  jax.ShapeDtypeStruct((_NC, _NP), _f32)],
        mesh=plsc.VectorSubcoreMesh(core_axis_name="c", subcore_axis_name="s"),
        scratch_types=(
            [pltpu.VMEM((2, _CH), jnp.int32)] * 4
            + [pltpu.VMEM((_CH, _H), _f32)] * 4
            + [pltpu.VMEM((32, 128), _f32)] * 2
            + [pltpu.VMEM((_CH, _H), _f32)] * 2
            + [pltpu.VMEM((_CH,), _f32),
               pltpu.VMEM((_RPT, _H), _f32),
               pltpu.VMEM((_RPT,), _f32),
               pltpu.VMEM_SHARED((_NP, _H), _f32),
               pltpu.VMEM_SHARED((_NP,), _f32)]
            + [pltpu.SemaphoreType.DMA] * 8
        ),
        compiler_params=pltpu.CompilerParams(use_tc_tiling_on_sc=False),
    )(_sc_edge_body)

    hp, cp = sc_edge(p, q, a0, a1, ei_v)

    out = pl.pallas_call(
        _post_body,
        grid=(_N // _BN,),
        in_specs=[
            pl.BlockSpec((_BN, _D), lambda i: (i, 0)),
            pl.BlockSpec((_NC, _BN, _H), lambda i: (0, i, 0)),
            pl.BlockSpec((_NC, _BN, 1), lambda i: (0, i, 0)),
            pl.BlockSpec((_H, _H), lambda i: (0, 0)),
            pl.BlockSpec((1, _H), lambda i: (0, 0)),
            pl.BlockSpec((_D, _D), lambda i: (0, 0)),
            pl.BlockSpec((_H, _D), lambda i: (0, 0)),
            pl.BlockSpec((1, _D), lambda i: (0, 0)),
        ],
        out_specs=pl.BlockSpec((_BN, _D), lambda i: (i, 0)),
        out_shape=jax.ShapeDtypeStruct((_N, _D), _f32),
    )(x, hp, cp.reshape(_NC, _NP, 1), w2t, b2r, w3at, w3bt, b3r)

    return out


# speedup vs baseline: 13.1952x; 1.0689x over previous
import functools

import jax
import jax.numpy as jnp
from jax import lax
from jax.experimental import pallas as pl
from jax.experimental.pallas import tpu as pltpu
from jax.experimental.pallas import tpu_sc as plsc

_N, _E, _D, _DE, _H = 10000, 320000, 128, 16, 32

_NC, _NS = 2, 16
_NW = _NC * _NS
_CH = 128
_EC = _E // _CH
_JMAX = (_EC + _NW - 1) // _NW
_NP = 10240
_RPT = _NP // _NS

_f32 = jnp.float32


def _pre_nodes_body(x_ref, w1at_ref, w1bt_ref, b1_ref, p_ref, q_ref):
    xb = x_ref[...]
    p_ref[...] = jnp.dot(xb, w1at_ref[...], preferred_element_type=_f32)
    q_ref[...] = (jnp.dot(xb, w1bt_ref[...], preferred_element_type=_f32)
                  + b1_ref[...])


_BEB = 12800
_B8 = _BEB // 8
_CPB = _BEB // _CH


def _pre_edges_body(eat_ref, wk_ref, a0_ref, a1_ref):
    eb = eat_ref[...].astype(jnp.bfloat16)

    def part(k):
        return lax.dot_general(eb[:, k * _B8:(k + 1) * _B8], wk_ref[k % 4],
                               (((0,), (0,)), ((), ())),
                               preferred_element_type=_f32)
    a0_ref[...] = part(0) + part(1) + part(2) + part(3)
    a1_ref[...] = part(4) + part(5) + part(6) + part(7)


def _sc_edge_body(p_hbm, q_hbm, a0_hbm, a1_hbm, ei_hbm,
                  hp_hbm, cp_hbm,
                  sivd0, sivd1, sivd2, sivd3,
                  pbuf0, qbuf0, pbuf1, qbuf1,
                  abuf0, abuf1, hbuf0, hbuf1,
                  onesv, zbuf, zcnt, acc, cacc,
                  semidx0, semidx1, sema0, sema1,
                  semg0, semg1, semsc0, semsc1):
    cid = lax.axis_index("c")
    sid = lax.axis_index("s")
    wid = sid * _NC + cid
    zero16 = jnp.zeros((16,), _f32)
    one16 = jnp.ones((16,), _f32)

    ibufs = (sivd0, sivd1, sivd2, sivd3)
    cbufs = ((pbuf0, qbuf0, abuf0, hbuf0, semidx0, sema0, semg0, semsc0),
             (pbuf1, qbuf1, abuf1, hbuf1, semidx1, sema1, semg1, semsc1))

    def _zrow(r, c):
        zbuf[r, 0:16] = zero16
        zbuf[r, 16:32] = zero16
        return c
    lax.fori_loop(0, _RPT, _zrow, 0)

    def _zcnt(i, c):
        zcnt[pl.ds(i * 16, 16)] = zero16
        return c
    lax.fori_loop(0, _RPT // 16, _zcnt, 0)

    def _ones(i, c):
        onesv[pl.ds(i * 16, 16)] = one16
        return c
    lax.fori_loop(0, _CH // 16, _ones, 0)

    pltpu.sync_copy(zbuf, acc.at[pl.ds(sid * _RPT, _RPT)])
    pltpu.sync_copy(zcnt, cacc.at[pl.ds(sid * _RPT, _RPT)])
    plsc.subcore_barrier()

    def issue_idx(cidx, I, sem):
        b = (cidx * 10486) >> 20
        ebase = _BEB * b + 16 * (cidx - _CPB * b)
        for half in range(2):
            for k in range(4):
                off = ebase + 4 * _B8 * half + _B8 * k
                pos = 16 * (4 * half + k)
                c0 = pl.multiple_of(off & 127, 16)
                pltpu.async_copy(ei_hbm.at[off >> 7, slice(None),
                                           pl.ds(c0, 16)],
                                 I.at[:, pl.ds(pos, 16)], sem)

    def drain_idx(I, sem):
        pltpu.make_async_copy(ei_hbm.at[0], I, sem).wait()

    def issue_a(cidx, C):
        abuf = C[2]
        pltpu.async_copy(a0_hbm.at[pl.ds(cidx * 16, 16)],
                         abuf.at[pl.ds(0, 16)], C[5])
        pltpu.async_copy(a1_hbm.at[pl.ds(cidx * 16, 16)],
                         abuf.at[pl.ds(16, 16)], C[5])

    def drain_a(C):
        abuf = C[2]
        pltpu.make_async_copy(a0_hbm.at[pl.ds(0, 16)],
                              abuf.at[pl.ds(0, 16)], C[5]).wait()
        pltpu.make_async_copy(a1_hbm.at[pl.ds(0, 16)],
                              abuf.at[pl.ds(16, 16)], C[5]).wait()

    def issue_gather(I, C):
        pltpu.async_copy(p_hbm.at[I.at[0]], C[0], C[6])
        pltpu.async_copy(q_hbm.at[I.at[1]], C[1], C[6])

    def drain_gather(C):
        pltpu.make_async_copy(p_hbm.at[pl.ds(0, _CH)], C[0], C[6]).wait()
        pltpu.make_async_copy(q_hbm.at[pl.ds(0, _CH)], C[1], C[6]).wait()

    def issue_scatter(I, C):
        pltpu.async_copy(C[3], acc.at[I.at[1]], C[7], add=True)
        pltpu.async_copy(onesv, cacc.at[I.at[1]], C[7], add=True)

    def drain_scatter(I, C):
        pltpu.make_async_copy(C[3], acc.at[I.at[1]], C[7]).wait()
        pltpu.make_async_copy(onesv, cacc.at[I.at[1]], C[7]).wait()

    def valu(C):
        pbuf, qbuf, abuf, hbuf = C[0], C[1], C[2], C[3]

        def _row(i, cc):
            for half in range(2):
                r = 16 * half + i
                for k in range(4):
                    e = 64 * half + 16 * k + i
                    for hh in range(2):
                        col = 32 * k + 16 * hh
                        v = (pbuf[e, pl.ds(16 * hh, 16)]
                             + qbuf[e, pl.ds(16 * hh, 16)]
                             + abuf[r, pl.ds(col, 16)])
                        hbuf[e, pl.ds(16 * hh, 16)] = jnp.maximum(v, 0.0)
            return cc
        lax.fori_loop(0, 16, _row, 0)

    issue_idx(wid, ibufs[0], cbufs[0][4])
    issue_a(wid, cbufs[0])
    drain_idx(ibufs[0], cbufs[0][4])
    issue_gather(ibufs[0], cbufs[0])
    issue_idx(_NW + wid, ibufs[1], cbufs[1][4])
    issue_a(_NW + wid, cbufs[1])

    def _quad(jj, c):
        for par in range(4):
            par2 = par % 2
            C = cbufs[par2]
            Cn = cbufs[1 - par2]
            I = ibufs[par]
            In1 = ibufs[(par + 1) % 4]
            In2 = ibufs[(par + 2) % 4]
            Im2 = ibufs[(par + 2) % 4]
            s = 4 * jj + par
            cidx = s * _NW + wid
            act_s = cidx < _EC
            act_s1 = cidx + _NW < _EC
            act_s2 = cidx + 2 * _NW < _EC

            @pl.when(act_s)
            def _():
                drain_gather(C)
                drain_a(C)

            @pl.when(s >= 2)
            def _():
                drain_scatter(Im2, C)

            @pl.when(act_s1)
            def _():
                drain_idx(In1, Cn[4])
                issue_gather(In1, Cn)

            @pl.when(act_s2)
            def _():
                issue_idx(cidx + 2 * _NW, In2, C[4])

            @pl.when(act_s)
            def _():
                valu(C)

            @pl.when(act_s2)
            def _():
                issue_a(cidx + 2 * _NW, C)

            @pl.when(act_s)
            def _():
                issue_scatter(I, C)
        return c

    lax.fori_loop(0, 20, _quad, 0)

    @pl.when(78 * _NW + wid < _EC)
    def _():
        drain_scatter(ibufs[2], cbufs[0])

    plsc.subcore_barrier()

    pltpu.sync_copy(acc.at[pl.ds(sid * _RPT, _RPT)],
                    hp_hbm.at[cid, pl.ds(sid * _RPT, _RPT)])
    pltpu.sync_copy(cacc.at[pl.ds(sid * _RPT, _RPT)],
                    cp_hbm.at[cid, pl.ds(sid * _RPT, _RPT)])


def _post_body(x_ref, hp_ref, cp_ref, w2t_ref, b2_ref, w3at_ref, w3bt_ref,
               b3_ref, o_ref):
    hagg = hp_ref[0] + hp_ref[1]
    cnt = cp_ref[0] + cp_ref[1]
    agg = ((jnp.dot(hagg, w2t_ref[...], preferred_element_type=_f32)
            + cnt * b2_ref[...]) / (cnt + 1e-6))
    xb = x_ref[...]
    up = (jnp.dot(xb, w3at_ref[...], preferred_element_type=_f32)
          + jnp.dot(agg, w3bt_ref[...], preferred_element_type=_f32)
          + b3_ref[...])
    o_ref[...] = xb + jnp.maximum(up, 0.0)


_BN = 1000
_BE = 8000


def kernel(x, edge_index, edge_attr, W1, b1, W2, b2, W3, b3):
    w1at = W1[:, :_D].T
    w1bt = W1[:, _D:2 * _D].T
    w1ct = W1[:, 2 * _D:].T
    w2t = W2.T
    w3at = W3[:, :_D].T
    w3bt = W3[:, _D:].T
    b1r = b1.reshape(1, _H)
    b2r = b2.reshape(1, _H)
    b3r = b3.reshape(1, _D)
    ei_v = edge_index.reshape(2, _E // 128, 128).transpose(1, 0, 2)
    eat = edge_attr.T
    wk = jnp.stack([jnp.zeros((_DE, 128), _f32)
                    .at[:, 32 * k:32 * (k + 1)].set(w1ct)
                    for k in range(4)]).astype(jnp.bfloat16)

    p, q = pl.pallas_call(
        _pre_nodes_body,
        grid=(_N // _BN,),
        in_specs=[
            pl.BlockSpec((_BN, _D), lambda i: (i, 0)),
            pl.BlockSpec((_D, _H), lambda i: (0, 0)),
            pl.BlockSpec((_D, _H), lambda i: (0, 0)),
            pl.BlockSpec((1, _H), lambda i: (0, 0)),
        ],
        out_specs=[pl.BlockSpec((_BN, _H), lambda i: (i, 0)),
                   pl.BlockSpec((_BN, _H), lambda i: (i, 0))],
        out_shape=[jax.ShapeDtypeStruct((_N, _H), _f32),
                   jax.ShapeDtypeStruct((_N, _H), _f32)],
    )(x, w1at, w1bt, b1r)

    a0, a1 = pl.pallas_call(
        _pre_edges_body,
        grid=(_E // _BEB,),
        in_specs=[
            pl.BlockSpec((_DE, _BEB), lambda i: (0, i)),
            pl.BlockSpec((4, _DE, 128), lambda i: (0, 0, 0)),
        ],
        out_specs=[pl.BlockSpec((_B8, 128), lambda i: (i, 0)),
                   pl.BlockSpec((_B8, 128), lambda i: (i, 0))],
        out_shape=[jax.ShapeDtypeStruct((_E // 8, 128), _f32),
                   jax.ShapeDtypeStruct((_E // 8, 128), _f32)],
        compiler_params=pltpu.CompilerParams(
            fuse_transposed_lhs_in_matmul=True),
    )(eat, wk)

    sc_edge = functools.partial(
        pl.kernel,
        out_type=[jax.ShapeDtypeStruct((_NC, _NP, _H), _f32),
                  jax.ShapeDtypeStruct((_NC, _NP), _f32)],
        mesh=plsc.VectorSubcoreMesh(core_axis_name="c", subcore_axis_name="s"),
        scratch_types=(
            [pltpu.VMEM((2, _CH), jnp.int32)] * 4
            + [pltpu.VMEM((_CH, _H), _f32)] * 4
            + [pltpu.VMEM((32, 128), _f32)] * 2
            + [pltpu.VMEM((_CH, _H), _f32)] * 2
            + [pltpu.VMEM((_CH,), _f32),
               pltpu.VMEM((_RPT, _H), _f32),
               pltpu.VMEM((_RPT,), _f32),
               pltpu.VMEM_SHARED((_NP, _H), _f32),
               pltpu.VMEM_SHARED((_NP,), _f32)]
            + [pltpu.SemaphoreType.DMA] * 8
        ),
        compiler_params=pltpu.CompilerParams(use_tc_tiling_on_sc=False),
    )(_sc_edge_body)

    hp, cp = sc_edge(p, q, a0, a1, ei_v)

    out = pl.pallas_call(
        _post_body,
        grid=(_N // _BN,),
        in_specs=[
            pl.BlockSpec((_BN, _D), lambda i: (i, 0)),
            pl.BlockSpec((_NC, _BN, _H), lambda i: (0, i, 0)),
            pl.BlockSpec((_NC, _BN, 1), lambda i: (0, i, 0)),
            pl.BlockSpec((_H, _H), lambda i: (0, 0)),
            pl.BlockSpec((1, _H), lambda i: (0, 0)),
            pl.BlockSpec((_D, _D), lambda i: (0, 0)),
            pl.BlockSpec((_H, _D), lambda i: (0, 0)),
            pl.BlockSpec((1, _D), lambda i: (0, 0)),
        ],
        out_specs=pl.BlockSpec((_BN, _D), lambda i: (i, 0)),
        out_shape=jax.ShapeDtypeStruct((_N, _D), _f32),
    )(x, hp, cp.reshape(_NC, _NP, 1), w2t, b2r, w3at, w3bt, b3r)

    return out


# speedup vs baseline: 13.3408x; 1.0110x over previous
import functools

import jax
import jax.numpy as jnp
from jax import lax
from jax.experimental import pallas as pl
from jax.experimental.pallas import tpu as pltpu
from jax.experimental.pallas import tpu_sc as plsc

_N, _E, _D, _DE, _H = 10000, 320000, 128, 16, 32

_NC, _NS = 2, 16
_NW = _NC * _NS
_CH = 256
_EC = _E // _CH
_NSLOT = (((_EC + _NW - 1) // _NW + 3) // 4) * 4
_NP = 10240
_RPT = _NP // _NS

_f32 = jnp.float32


def _pre_nodes_body(x_ref, w1at_ref, w1bt_ref, b1_ref, p_ref, q_ref):
    xb = x_ref[...]
    p_ref[...] = jnp.dot(xb, w1at_ref[...], preferred_element_type=_f32)
    q_ref[...] = (jnp.dot(xb, w1bt_ref[...], preferred_element_type=_f32)
                  + b1_ref[...])


_BEB = 12800
_B8 = _BEB // 8
_CPB = _BEB // _CH


def _pre_edges_body(eat_ref, wk_ref, a0_ref, a1_ref):
    eb = eat_ref[...].astype(jnp.bfloat16)

    def part(k):
        return lax.dot_general(eb[:, k * _B8:(k + 1) * _B8], wk_ref[k % 4],
                               (((0,), (0,)), ((), ())),
                               preferred_element_type=_f32)
    a0_ref[...] = part(0) + part(1) + part(2) + part(3)
    a1_ref[...] = part(4) + part(5) + part(6) + part(7)


def _sc_edge_body(p_hbm, q_hbm, a0_hbm, a1_hbm, ei_hbm,
                  hp_hbm, cp_hbm,
                  sivd0, sivd1, sivd2, sivd3,
                  pbuf0, qbuf0, pbuf1, qbuf1,
                  abuf0, abuf1, hbuf0, hbuf1,
                  onesv, zbuf, zcnt, acc, cacc,
                  semidx0, semidx1, sema0, sema1,
                  semg0, semg1, semsc0, semsc1):
    cid = lax.axis_index("c")
    sid = lax.axis_index("s")
    wid = sid * _NC + cid
    zero16 = jnp.zeros((16,), _f32)
    one16 = jnp.ones((16,), _f32)

    ibufs = (sivd0, sivd1, sivd2, sivd3)
    cbufs = ((pbuf0, qbuf0, abuf0, hbuf0, semidx0, sema0, semg0, semsc0),
             (pbuf1, qbuf1, abuf1, hbuf1, semidx1, sema1, semg1, semsc1))

    def _zrow(r, c):
        zbuf[r, 0:16] = zero16
        zbuf[r, 16:32] = zero16
        return c
    lax.fori_loop(0, _RPT, _zrow, 0)

    def _zcnt(i, c):
        zcnt[pl.ds(i * 16, 16)] = zero16
        return c
    lax.fori_loop(0, _RPT // 16, _zcnt, 0)

    def _ones(i, c):
        onesv[pl.ds(i * 16, 16)] = one16
        return c
    lax.fori_loop(0, _CH // 16, _ones, 0)

    pltpu.sync_copy(zbuf, acc.at[pl.ds(sid * _RPT, _RPT)])
    pltpu.sync_copy(zcnt, cacc.at[pl.ds(sid * _RPT, _RPT)])
    plsc.subcore_barrier()

    _R = _CH // 8

    def issue_idx(cidx, I, sem):
        b = (cidx * 10486) >> 19
        ebase = _BEB * b + _R * (cidx - _CPB * b)
        for half in range(2):
            for k in range(4):
                off = ebase + 4 * _B8 * half + _B8 * k
                pos = _R * (4 * half + k)
                c0 = pl.multiple_of(off & 127, _R)
                pltpu.async_copy(ei_hbm.at[off >> 7, slice(None),
                                           pl.ds(c0, _R)],
                                 I.at[:, pl.ds(pos, _R)], sem)

    def drain_idx(I, sem):
        pltpu.make_async_copy(ei_hbm.at[0], I, sem).wait()

    _AR = _CH // 8

    def issue_a(cidx, C):
        abuf = C[2]
        pltpu.async_copy(a0_hbm.at[pl.ds(cidx * _AR, _AR)],
                         abuf.at[pl.ds(0, _AR)], C[5])
        pltpu.async_copy(a1_hbm.at[pl.ds(cidx * _AR, _AR)],
                         abuf.at[pl.ds(_AR, _AR)], C[5])

    def drain_a(C):
        abuf = C[2]
        pltpu.make_async_copy(a0_hbm.at[pl.ds(0, _AR)],
                              abuf.at[pl.ds(0, _AR)], C[5]).wait()
        pltpu.make_async_copy(a1_hbm.at[pl.ds(0, _AR)],
                              abuf.at[pl.ds(_AR, _AR)], C[5]).wait()

    def issue_gather(I, C):
        pltpu.async_copy(p_hbm.at[I.at[0]], C[0], C[6])
        pltpu.async_copy(q_hbm.at[I.at[1]], C[1], C[6])

    def drain_gather(C):
        pltpu.make_async_copy(p_hbm.at[pl.ds(0, _CH)], C[0], C[6]).wait()
        pltpu.make_async_copy(q_hbm.at[pl.ds(0, _CH)], C[1], C[6]).wait()

    def issue_scatter(I, C):
        pltpu.async_copy(C[3], acc.at[I.at[1]], C[7], add=True)
        pltpu.async_copy(onesv, cacc.at[I.at[1]], C[7], add=True)

    def drain_scatter(I, C):
        pltpu.make_async_copy(C[3], acc.at[I.at[1]], C[7]).wait()
        pltpu.make_async_copy(onesv, cacc.at[I.at[1]], C[7]).wait()

    def valu(C):
        pbuf, qbuf, abuf, hbuf = C[0], C[1], C[2], C[3]

        def _row(i, cc):
            for half in range(2):
                r = (_CH // 8) * half + i
                for k in range(4):
                    e = (_CH // 2) * half + (_CH // 8) * k + i
                    for hh in range(2):
                        col = 32 * k + 16 * hh
                        v = (pbuf[e, pl.ds(16 * hh, 16)]
                             + qbuf[e, pl.ds(16 * hh, 16)]
                             + abuf[r, pl.ds(col, 16)])
                        hbuf[e, pl.ds(16 * hh, 16)] = jnp.maximum(v, 0.0)
            return cc
        lax.fori_loop(0, _CH // 8, _row, 0)

    issue_idx(wid, ibufs[0], cbufs[0][4])
    issue_a(wid, cbufs[0])
    drain_idx(ibufs[0], cbufs[0][4])
    issue_gather(ibufs[0], cbufs[0])
    issue_idx(_NW + wid, ibufs[1], cbufs[1][4])
    issue_a(_NW + wid, cbufs[1])

    def _quad(jj, c):
        for par in range(4):
            par2 = par % 2
            C = cbufs[par2]
            Cn = cbufs[1 - par2]
            I = ibufs[par]
            In1 = ibufs[(par + 1) % 4]
            In2 = ibufs[(par + 2) % 4]
            Im2 = ibufs[(par + 2) % 4]
            s = 4 * jj + par
            cidx = s * _NW + wid
            act_s = cidx < _EC
            act_s1 = cidx + _NW < _EC
            act_s2 = cidx + 2 * _NW < _EC

            @pl.when(act_s)
            def _():
                drain_gather(C)
                drain_a(C)

            @pl.when(s >= 2)
            def _():
                drain_scatter(Im2, C)

            @pl.when(act_s1)
            def _():
                drain_idx(In1, Cn[4])
                issue_gather(In1, Cn)

            @pl.when(act_s2)
            def _():
                issue_idx(cidx + 2 * _NW, In2, C[4])

            @pl.when(act_s)
            def _():
                valu(C)

            @pl.when(act_s2)
            def _():
                issue_a(cidx + 2 * _NW, C)

            @pl.when(act_s)
            def _():
                issue_scatter(I, C)
        return c

    lax.fori_loop(0, _NSLOT // 4, _quad, 0)

    drain_scatter(ibufs[(_NSLOT - 2) % 4], cbufs[(_NSLOT - 2) % 2])

    @pl.when((_NSLOT - 1) * _NW + wid < _EC)
    def _():
        drain_scatter(ibufs[(_NSLOT - 1) % 4], cbufs[(_NSLOT - 1) % 2])

    plsc.subcore_barrier()

    pltpu.sync_copy(acc.at[pl.ds(sid * _RPT, _RPT)],
                    hp_hbm.at[cid, pl.ds(sid * _RPT, _RPT)])
    pltpu.sync_copy(cacc.at[pl.ds(sid * _RPT, _RPT)],
                    cp_hbm.at[cid, pl.ds(sid * _RPT, _RPT)])


def _post_body(x_ref, hp_ref, cp_ref, w2t_ref, b2_ref, w3at_ref, w3bt_ref,
               b3_ref, o_ref):
    hagg = hp_ref[0] + hp_ref[1]
    cnt = cp_ref[0] + cp_ref[1]
    agg = ((jnp.dot(hagg, w2t_ref[...], preferred_element_type=_f32)
            + cnt * b2_ref[...]) / (cnt + 1e-6))
    xb = x_ref[...]
    up = (jnp.dot(xb, w3at_ref[...], preferred_element_type=_f32)
          + jnp.dot(agg, w3bt_ref[...], preferred_element_type=_f32)
          + b3_ref[...])
    o_ref[...] = xb + jnp.maximum(up, 0.0)


_BN = 1000
_BE = 8000


def kernel(x, edge_index, edge_attr, W1, b1, W2, b2, W3, b3):
    w1at = W1[:, :_D].T
    w1bt = W1[:, _D:2 * _D].T
    w1ct = W1[:, 2 * _D:].T
    w2t = W2.T
    w3at = W3[:, :_D].T
    w3bt = W3[:, _D:].T
    b1r = b1.reshape(1, _H)
    b2r = b2.reshape(1, _H)
    b3r = b3.reshape(1, _D)
    ei_v = edge_index.reshape(2, _E // 128, 128).transpose(1, 0, 2)
    eat = edge_attr.T
    wk = jnp.stack([jnp.zeros((_DE, 128), _f32)
                    .at[:, 32 * k:32 * (k + 1)].set(w1ct)
                    for k in range(4)]).astype(jnp.bfloat16)

    p, q = pl.pallas_call(
        _pre_nodes_body,
        grid=(_N // _BN,),
        in_specs=[
            pl.BlockSpec((_BN, _D), lambda i: (i, 0)),
            pl.BlockSpec((_D, _H), lambda i: (0, 0)),
            pl.BlockSpec((_D, _H), lambda i: (0, 0)),
            pl.BlockSpec((1, _H), lambda i: (0, 0)),
        ],
        out_specs=[pl.BlockSpec((_BN, _H), lambda i: (i, 0)),
                   pl.BlockSpec((_BN, _H), lambda i: (i, 0))],
        out_shape=[jax.ShapeDtypeStruct((_N, _H), _f32),
                   jax.ShapeDtypeStruct((_N, _H), _f32)],
    )(x, w1at, w1bt, b1r)

    a0, a1 = pl.pallas_call(
        _pre_edges_body,
        grid=(_E // _BEB,),
        in_specs=[
            pl.BlockSpec((_DE, _BEB), lambda i: (0, i)),
            pl.BlockSpec((4, _DE, 128), lambda i: (0, 0, 0)),
        ],
        out_specs=[pl.BlockSpec((_B8, 128), lambda i: (i, 0)),
                   pl.BlockSpec((_B8, 128), lambda i: (i, 0))],
        out_shape=[jax.ShapeDtypeStruct((_E // 8, 128), _f32),
                   jax.ShapeDtypeStruct((_E // 8, 128), _f32)],
        compiler_params=pltpu.CompilerParams(
            fuse_transposed_lhs_in_matmul=True),
    )(eat, wk)

    sc_edge = functools.partial(
        pl.kernel,
        out_type=[jax.ShapeDtypeStruct((_NC, _NP, _H), _f32),
                  jax.ShapeDtypeStruct((_NC, _NP), _f32)],
        mesh=plsc.VectorSubcoreMesh(core_axis_name="c", subcore_axis_name="s"),
        scratch_types=(
            [pltpu.VMEM((2, _CH), jnp.int32)] * 4
            + [pltpu.VMEM((_CH, _H), _f32)] * 4
            + [pltpu.VMEM((_CH // 4, 128), _f32)] * 2
            + [pltpu.VMEM((_CH, _H), _f32)] * 2
            + [pltpu.VMEM((_CH,), _f32),
               pltpu.VMEM((_RPT, _H), _f32),
               pltpu.VMEM((_RPT,), _f32),
               pltpu.VMEM_SHARED((_NP, _H), _f32),
               pltpu.VMEM_SHARED((_NP,), _f32)]
            + [pltpu.SemaphoreType.DMA] * 8
        ),
        compiler_params=pltpu.CompilerParams(use_tc_tiling_on_sc=False),
    )(_sc_edge_body)

    hp, cp = sc_edge(p, q, a0, a1, ei_v)

    out = pl.pallas_call(
        _post_body,
        grid=(_N // _BN,),
        in_specs=[
            pl.BlockSpec((_BN, _D), lambda i: (i, 0)),
            pl.BlockSpec((_NC, _BN, _H), lambda i: (0, i, 0)),
            pl.BlockSpec((_NC, _BN, 1), lambda i: (0, i, 0)),
            pl.BlockSpec((_H, _H), lambda i: (0, 0)),
            pl.BlockSpec((1, _H), lambda i: (0, 0)),
            pl.BlockSpec((_D, _D), lambda i: (0, 0)),
            pl.BlockSpec((_H, _D), lambda i: (0, 0)),
            pl.BlockSpec((1, _D), lambda i: (0, 0)),
        ],
        out_specs=pl.BlockSpec((_BN, _D), lambda i: (i, 0)),
        out_shape=jax.ShapeDtypeStruct((_N, _D), _f32),
    )(x, hp, cp.reshape(_NC, _NP, 1), w2t, b2r, w3at, w3bt, b3r)

    return out
